# baseline (speedup 1.0000x reference)
import functools

import jax
import jax.numpy as jnp
from jax import lax
from jax.experimental import pallas as pl
from jax.experimental.pallas import tpu as pltpu
from jax.experimental.pallas import tpu_sc as plsc

N_NODES = 10000
IN_CH = 128
HID = 128
ROW_BLK = 1000

N_EDGES = 320000
NC = 2
NS = 16
NW = NC * NS
EPW = N_EDGES // NW
CHUNK = 64
NBUF = 4
EPW_PAD = 10240
NCHUNK = EPW_PAD // CHUNK
NSEG = 5
SEGCHUNK = NCHUNK // NSEG
NQUAD = SEGCHUNK // NBUF
ZBLK = 64
NZFULL = N_NODES // ZBLK

_sc_mesh = plsc.VectorSubcoreMesh(core_axis_name="c", subcore_axis_name="s")


@functools.partial(
    pl.kernel,
    out_type=jax.ShapeDtypeStruct((NC, N_NODES, HID), jnp.float32),
    mesh=_sc_mesh,
    scratch_types=[
        pltpu.VMEM((SEGCHUNK, CHUNK), jnp.int32),
        pltpu.VMEM((SEGCHUNK, CHUNK), jnp.int32),
        pltpu.VMEM((SEGCHUNK, CHUNK), jnp.float32),
        [pltpu.VMEM((CHUNK, HID), jnp.float32) for _ in range(NBUF)],
        pltpu.VMEM_SHARED((N_NODES, HID), jnp.float32),
        [pltpu.SemaphoreType.DMA for _ in range(NBUF)],
        [pltpu.SemaphoreType.DMA for _ in range(NBUF)],
    ],
)
def _lap_sc(v_hbm, src_hbm, dst_hbm, w_hbm, out_hbm,
            src_t, dst_t, w_t, rows, acc, gsem, ssem):
    cid = lax.axis_index("c")
    sid = lax.axis_index("s")
    wid = sid * NC + cid

    zeros16 = jnp.zeros((16,), jnp.float32)

    def zrow(i, _):
        for k in range(HID // 16):
            rows[0][i, pl.ds(k * 16, 16)] = zeros16
        return 0

    lax.fori_loop(0, ZBLK, zrow, 0)
    for j in range((NZFULL + NS) // NS):
        b = sid + j * NS

        @pl.when(b < NZFULL)
        def _():
            off = pl.multiple_of(b * ZBLK, ZBLK)
            pltpu.sync_copy(rows[0], acc.at[pl.ds(off, ZBLK)])

        @pl.when(b == NZFULL)
        def _():
            pltpu.sync_copy(rows[0].at[pl.ds(0, 16)],
                            acc.at[pl.ds(NZFULL * ZBLK, 16)])

    plsc.subcore_barrier()

    def g_issue(c, buf):
        pltpu.async_copy(v_hbm.at[src_t.at[c]], rows[buf], gsem[buf])

    def g_wait(buf):
        pltpu.make_async_copy(v_hbm.at[src_t.at[0]], rows[buf],
                              gsem[buf]).wait()

    def s_issue(c, buf):
        pltpu.async_copy(rows[buf], acc.at[dst_t.at[c]], ssem[buf], add=True)

    def s_wait(buf):
        pltpu.make_async_copy(rows[0], acc.at[dst_t.at[0]],
                              ssem[buf]).wait()

    def scale(c, buf):
        def group_body(g, _):
            wv = w_t[c, pl.ds(g * 16, 16)]
            for j in range(16):
                we = wv[j]
                e = g * 16 + j
                for k in range(HID // 16):
                    sl = pl.ds(k * 16, 16)
                    rows[buf][e, sl] = rows[buf][e, sl] * we
            return 0

        lax.fori_loop(0, CHUNK // 16, group_body, 0)

    for seg in range(NSEG):
        pltpu.sync_copy(src_hbm.at[wid, seg], src_t)
        pltpu.sync_copy(dst_hbm.at[wid, seg], dst_t)
        pltpu.sync_copy(w_hbm.at[wid, seg], w_t)

        g_issue(0, 0)
        g_issue(1, 1)

        def quad_body(q, _):
            for p in range(NBUF):
                c = q * NBUF + p
                g_wait(p)
                scale(c, p)

                @pl.when(c >= 2)
                def _():
                    s_wait((p + 2) % NBUF)

                @pl.when(c < SEGCHUNK - 2)
                def _():
                    g_issue(c + 2, (p + 2) % NBUF)

                s_issue(c, p)
            return 0

        lax.fori_loop(0, NQUAD, quad_body, 0)
        s_wait((SEGCHUNK - 2) % NBUF)
        s_wait((SEGCHUNK - 1) % NBUF)

    plsc.subcore_barrier()

    for j in range((NZFULL + NS) // NS):
        b = sid + j * NS

        @pl.when(b < NZFULL)
        def _():
            off = pl.multiple_of(b * ZBLK, ZBLK)
            pltpu.sync_copy(acc.at[pl.ds(off, ZBLK)],
                            out_hbm.at[cid, pl.ds(off, ZBLK)])

        @pl.when(b == NZFULL)
        def _():
            pltpu.sync_copy(acc.at[pl.ds(NZFULL * ZBLK, 16)],
                            out_hbm.at[cid, pl.ds(NZFULL * ZBLK, 16)])


def _gru_zr_body(gxzr_ref, hmm_ref, h_ref, z_ref, hr_ref):
    zr = jax.nn.sigmoid(gxzr_ref[...] + hmm_ref[...])
    z = zr[:, :HID]
    r = zr[:, HID:]
    z_ref[...] = z
    hr_ref[...] = h_ref[...] * r


def _gru_h_body(gxh_ref, hrmm_ref, z_ref, h_ref, out_ref):
    htil = jnp.tanh(gxh_ref[...] + hrmm_ref[...])
    z = z_ref[...]
    out_ref[...] = z * h_ref[...] + (1.0 - z) * htil


def _row_spec(c):
    return pl.BlockSpec((ROW_BLK, c), lambda i: (i, 0))


_gru_zr = pl.pallas_call(
    _gru_zr_body,
    grid=(N_NODES // ROW_BLK,),
    in_specs=[_row_spec(2 * HID), _row_spec(2 * HID), _row_spec(HID)],
    out_specs=[_row_spec(HID), _row_spec(HID)],
    out_shape=[
        jax.ShapeDtypeStruct((N_NODES, HID), jnp.float32),
        jax.ShapeDtypeStruct((N_NODES, HID), jnp.float32),
    ],
)

_gru_h = pl.pallas_call(
    _gru_h_body,
    grid=(N_NODES // ROW_BLK,),
    in_specs=[_row_spec(HID), _row_spec(HID), _row_spec(HID), _row_spec(HID)],
    out_specs=_row_spec(HID),
    out_shape=jax.ShapeDtypeStruct((N_NODES, HID), jnp.float32),
)


def _cat_weights(W):
    return jnp.concatenate([W[0] - W[2], W[1], 2.0 * W[2]], axis=0)


def kernel(X, edge_index, edge_weight, Wxz, bxz, Whz, bhz, Wxr, bxr, Whr, bhr,
           Wxh, bxh, Whh, bhh):
    src = edge_index[0]
    dst = edge_index[1]
    deg = jax.ops.segment_sum(edge_weight, src, num_segments=N_NODES)
    dinv = jnp.where(deg > 0, deg ** -0.5, 0.0)
    w = -dinv[src] * edge_weight * dinv[dst]

    pad_shape = (NW, EPW_PAD - EPW)
    src4 = jnp.concatenate(
        [src.reshape(NW, EPW), jnp.zeros(pad_shape, jnp.int32)], axis=1
    ).reshape(NW, NSEG, SEGCHUNK, CHUNK)
    dst4 = jnp.concatenate(
        [dst.reshape(NW, EPW), jnp.zeros(pad_shape, jnp.int32)], axis=1
    ).reshape(NW, NSEG, SEGCHUNK, CHUNK)
    w4 = jnp.concatenate(
        [w.reshape(NW, EPW), jnp.zeros(pad_shape, jnp.float32)], axis=1
    ).reshape(NW, NSEG, SEGCHUNK, CHUNK)

    def S(v):
        parts = _lap_sc(v, src4, dst4, w4)
        return parts[0] + parts[1]

    batches, seq_len, num_nodes, _ = X.shape

    Wx_cat = jnp.concatenate(
        [_cat_weights(Wxz), _cat_weights(Wxr), _cat_weights(Wxh)], axis=1)
    bx_cat = jnp.concatenate([bxz, bxr, bxh])
    Whzr_cat = jnp.concatenate([_cat_weights(Whz), _cat_weights(Whr)], axis=1)
    bh_zr = jnp.concatenate([bhz, bhr])
    Whh_cat = _cat_weights(Whh)

    Gx = []
    for t in range(seq_len):
        Xt = X[0, t]
        X1 = S(Xt)
        X2 = S(X1)
        basis = jnp.concatenate([Xt, X1, X2], axis=1)
        Gx.append(basis @ Wx_cat + bx_cat)

    H = jnp.zeros((num_nodes, HID), dtype=X.dtype)
    states = []
    for t in range(seq_len):
        H1 = S(H)
        H2 = S(H1)
        Bh = jnp.concatenate([H, H1, H2], axis=1)
        hmm = Bh @ Whzr_cat + bh_zr
        Z, HR = _gru_zr(Gx[t][:, :2 * HID], hmm, H)
        R1 = S(HR)
        R2 = S(R1)
        Bhr = jnp.concatenate([HR, R1, R2], axis=1)
        hrmm = Bhr @ Whh_cat + bhh
        H = _gru_h(Gx[t][:, 2 * HID:], hrmm, Z, H)
        states.append(H)

    out = jnp.stack(states, axis=0)[None]
    return out, H[None]


# speedup vs baseline: 1.2272x; 1.2272x over previous
import functools

import jax
import jax.numpy as jnp
from jax import lax
from jax.experimental import pallas as pl
from jax.experimental.pallas import tpu as pltpu
from jax.experimental.pallas import tpu_sc as plsc

N_NODES = 10000
IN_CH = 128
HID = 128
ROW_BLK = 1000

N_EDGES = 320000
NC = 2
NS = 16
NW = NC * NS
EPW = N_EDGES // NW
CHUNK = 128
EPW_PAD = 10240
NCHUNK = EPW_PAD // CHUNK
NSEG = 5
SEGCHUNK = NCHUNK // NSEG
ZBLK = 64
NZFULL = N_NODES // ZBLK
NPAD = 10240

_sc_mesh = plsc.VectorSubcoreMesh(core_axis_name="c", subcore_axis_name="s")


@functools.partial(
    pl.kernel,
    out_type=jax.ShapeDtypeStruct((NC, NPAD), jnp.float32),
    mesh=_sc_mesh,
    scratch_types=[
        pltpu.VMEM((SEGCHUNK, CHUNK), jnp.int32),
        pltpu.VMEM((SEGCHUNK, CHUNK), jnp.float32),
        pltpu.VMEM((NPAD // NS,), jnp.float32),
        pltpu.VMEM_SHARED((NPAD,), jnp.float32),
    ],
)
def _deg_sc(src_hbm, ew_hbm, out_hbm, src_t, ew_t, zbuf, bins):
    cid = lax.axis_index("c")
    sid = lax.axis_index("s")
    wid = sid * NC + cid
    stripe = NPAD // NS

    zeros16 = jnp.zeros((16,), jnp.float32)

    def zrow(i, _):
        zbuf[pl.ds(i * 16, 16)] = zeros16
        return 0

    lax.fori_loop(0, stripe // 16, zrow, 0)
    soff = pl.multiple_of(sid * stripe, stripe)
    pltpu.sync_copy(zbuf, bins.at[pl.ds(soff, stripe)])
    plsc.subcore_barrier()

    for seg in range(NSEG):
        pltpu.sync_copy(src_hbm.at[wid, seg], src_t)
        pltpu.sync_copy(ew_hbm.at[wid, seg], ew_t)

        def chunk_body(c, _):
            pltpu.sync_copy(ew_t.at[c], bins.at[src_t.at[c]], add=True)
            return 0

        lax.fori_loop(0, SEGCHUNK, chunk_body, 0)

    plsc.subcore_barrier()
    pltpu.sync_copy(bins.at[pl.ds(soff, stripe)],
                    out_hbm.at[cid, pl.ds(soff, stripe)])


@functools.partial(
    pl.kernel,
    out_type=jax.ShapeDtypeStruct((NW, NSEG, SEGCHUNK, CHUNK), jnp.float32),
    mesh=_sc_mesh,
    scratch_types=[
        pltpu.VMEM((SEGCHUNK, CHUNK), jnp.int32),
        pltpu.VMEM((SEGCHUNK, CHUNK), jnp.int32),
        pltpu.VMEM((SEGCHUNK, CHUNK), jnp.float32),
        pltpu.VMEM((CHUNK,), jnp.float32),
        pltpu.VMEM((CHUNK,), jnp.float32),
        pltpu.VMEM((SEGCHUNK, CHUNK), jnp.float32),
        pltpu.SemaphoreType.DMA,
        pltpu.SemaphoreType.DMA,
    ],
)
def _w_sc(dinv_hbm, src_hbm, dst_hbm, ew_hbm, w_hbm,
          src_t, dst_t, ew_t, dsrc_t, ddst_t, wout_t, sem1, sem2):
    cid = lax.axis_index("c")
    sid = lax.axis_index("s")
    wid = sid * NC + cid

    for seg in range(NSEG):
        pltpu.sync_copy(src_hbm.at[wid, seg], src_t)
        pltpu.sync_copy(dst_hbm.at[wid, seg], dst_t)
        pltpu.sync_copy(ew_hbm.at[wid, seg], ew_t)

        def chunk_body(c, _):
            d1 = pltpu.async_copy(dinv_hbm.at[src_t.at[c]], dsrc_t, sem1)
            d2 = pltpu.async_copy(dinv_hbm.at[dst_t.at[c]], ddst_t, sem2)
            d1.wait()
            d2.wait()

            def group_body(g, _):
                sl = pl.ds(g * 16, 16)
                wout_t[c, sl] = -(dsrc_t[sl] * ew_t[c, sl] * ddst_t[sl])
                return 0

            lax.fori_loop(0, CHUNK // 16, group_body, 0)
            return 0

        lax.fori_loop(0, SEGCHUNK, chunk_body, 0)
        pltpu.sync_copy(wout_t, w_hbm.at[wid, seg])


@functools.partial(
    pl.kernel,
    out_type=jax.ShapeDtypeStruct((NC, N_NODES, HID), jnp.float32),
    mesh=_sc_mesh,
    scratch_types=[
        pltpu.VMEM((SEGCHUNK, CHUNK), jnp.int32),
        pltpu.VMEM((SEGCHUNK, CHUNK), jnp.int32),
        pltpu.VMEM((SEGCHUNK, CHUNK), jnp.float32),
        pltpu.VMEM((CHUNK, HID), jnp.float32),
        pltpu.VMEM_SHARED((N_NODES, HID), jnp.float32),
        pltpu.SemaphoreType.DMA,
    ],
)
def _lap_sc(v_hbm, src_hbm, dst_hbm, w_hbm, out_hbm,
            src_t, dst_t, w_t, rows, acc, sem):
    cid = lax.axis_index("c")
    sid = lax.axis_index("s")
    wid = sid * NC + cid

    zeros16 = jnp.zeros((16,), jnp.float32)

    def zrow(i, _):
        for k in range(HID // 16):
            rows[i, pl.ds(k * 16, 16)] = zeros16
        return 0

    lax.fori_loop(0, ZBLK, zrow, 0)
    for j in range((NZFULL + NS) // NS):
        b = sid + j * NS

        @pl.when(b < NZFULL)
        def _():
            off = pl.multiple_of(b * ZBLK, ZBLK)
            pltpu.sync_copy(rows.at[pl.ds(0, ZBLK)], acc.at[pl.ds(off, ZBLK)])

        @pl.when(b == NZFULL)
        def _():
            pltpu.sync_copy(rows.at[pl.ds(0, 16)],
                            acc.at[pl.ds(NZFULL * ZBLK, 16)])

    plsc.subcore_barrier()

    for seg in range(NSEG):
        pltpu.sync_copy(src_hbm.at[wid, seg], src_t)
        pltpu.sync_copy(dst_hbm.at[wid, seg], dst_t)
        pltpu.sync_copy(w_hbm.at[wid, seg], w_t)

        def chunk_body(c, _):
            pltpu.async_copy(v_hbm.at[src_t.at[c]], rows, sem).wait()

            def group_body(g, _):
                wv = w_t[c, pl.ds(g * 16, 16)]
                for j in range(16):
                    we = wv[j]
                    e = g * 16 + j
                    for k in range(HID // 16):
                        sl = pl.ds(k * 16, 16)
                        rows[e, sl] = rows[e, sl] * we
                return 0

            lax.fori_loop(0, CHUNK // 16, group_body, 0)
            pltpu.sync_copy(rows, acc.at[dst_t.at[c]], add=True)
            return 0

        lax.fori_loop(0, SEGCHUNK, chunk_body, 0)

    plsc.subcore_barrier()

    for j in range((NZFULL + NS) // NS):
        b = sid + j * NS

        @pl.when(b < NZFULL)
        def _():
            off = pl.multiple_of(b * ZBLK, ZBLK)
            pltpu.sync_copy(acc.at[pl.ds(off, ZBLK)],
                            out_hbm.at[cid, pl.ds(off, ZBLK)])

        @pl.when(b == NZFULL)
        def _():
            pltpu.sync_copy(acc.at[pl.ds(NZFULL * ZBLK, 16)],
                            out_hbm.at[cid, pl.ds(NZFULL * ZBLK, 16)])


def _gru_zr_body(gxzr_ref, hmm_ref, h_ref, z_ref, hr_ref):
    zr = jax.nn.sigmoid(gxzr_ref[...] + hmm_ref[...])
    z = zr[:, :HID]
    r = zr[:, HID:]
    z_ref[...] = z
    hr_ref[...] = h_ref[...] * r


def _gru_h_body(gxh_ref, hrmm_ref, z_ref, h_ref, out_ref):
    htil = jnp.tanh(gxh_ref[...] + hrmm_ref[...])
    z = z_ref[...]
    out_ref[...] = z * h_ref[...] + (1.0 - z) * htil


def _row_spec(c):
    return pl.BlockSpec((ROW_BLK, c), lambda i: (i, 0))


_gru_zr = pl.pallas_call(
    _gru_zr_body,
    grid=(N_NODES // ROW_BLK,),
    in_specs=[_row_spec(2 * HID), _row_spec(2 * HID), _row_spec(HID)],
    out_specs=[_row_spec(HID), _row_spec(HID)],
    out_shape=[
        jax.ShapeDtypeStruct((N_NODES, HID), jnp.float32),
        jax.ShapeDtypeStruct((N_NODES, HID), jnp.float32),
    ],
)

_gru_h = pl.pallas_call(
    _gru_h_body,
    grid=(N_NODES // ROW_BLK,),
    in_specs=[_row_spec(HID), _row_spec(HID), _row_spec(HID), _row_spec(HID)],
    out_specs=_row_spec(HID),
    out_shape=jax.ShapeDtypeStruct((N_NODES, HID), jnp.float32),
)


def _cat_weights(W):
    return jnp.concatenate([W[0] - W[2], W[1], 2.0 * W[2]], axis=0)


def _edge_layout(a, fill):
    a = a.reshape(NW, EPW)
    pad = jnp.full((NW, EPW_PAD - EPW), fill, a.dtype)
    return jnp.concatenate([a, pad], axis=1).reshape(NW, NSEG, SEGCHUNK, CHUNK)


def kernel(X, edge_index, edge_weight, Wxz, bxz, Whz, bhz, Wxr, bxr, Whr, bhr,
           Wxh, bxh, Whh, bhh):
    src = edge_index[0]
    dst = edge_index[1]

    src4 = _edge_layout(src, 0)
    dst4 = _edge_layout(dst, 0)
    ew4 = _edge_layout(edge_weight, 0.0)

    degp = _deg_sc(src4, ew4)
    deg = degp[0] + degp[1]
    dinv = jnp.where(deg > 0, deg ** -0.5, 0.0)
    w4 = _w_sc(dinv, src4, dst4, ew4)

    def S(v):
        parts = _lap_sc(v, src4, dst4, w4)
        return parts[0] + parts[1]

    batches, seq_len, num_nodes, _ = X.shape

    Wx_cat = jnp.concatenate(
        [_cat_weights(Wxz), _cat_weights(Wxr), _cat_weights(Wxh)], axis=1)
    bx_cat = jnp.concatenate([bxz, bxr, bxh])
    Whzr_cat = jnp.concatenate([_cat_weights(Whz), _cat_weights(Whr)], axis=1)
    bh_zr = jnp.concatenate([bhz, bhr])
    Whh_cat = _cat_weights(Whh)

    Gx = []
    for t in range(seq_len):
        Xt = X[0, t]
        X1 = S(Xt)
        X2 = S(X1)
        basis = jnp.concatenate([Xt, X1, X2], axis=1)
        Gx.append(basis @ Wx_cat + bx_cat)

    H = jnp.zeros((num_nodes, HID), dtype=X.dtype)
    states = []
    for t in range(seq_len):
        H1 = S(H)
        H2 = S(H1)
        Bh = jnp.concatenate([H, H1, H2], axis=1)
        hmm = Bh @ Whzr_cat + bh_zr
        Z, HR = _gru_zr(Gx[t][:, :2 * HID], hmm, H)
        R1 = S(HR)
        R2 = S(R1)
        Bhr = jnp.concatenate([HR, R1, R2], axis=1)
        hrmm = Bhr @ Whh_cat + bhh
        H = _gru_h(Gx[t][:, 2 * HID:], hrmm, Z, H)
        states.append(H)

    out = jnp.stack(states, axis=0)[None]
    return out, H[None]


# speedup vs baseline: 2.3270x; 1.8962x over previous
import functools

import jax
import jax.numpy as jnp
from jax import lax
from jax.experimental import pallas as pl
from jax.experimental.pallas import tpu as pltpu
from jax.experimental.pallas import tpu_sc as plsc

N_NODES = 10000
IN_CH = 128
HID = 128
ROW_BLK = 1000

N_EDGES = 320000
NC = 2
NS = 16
NW = NC * NS
EPW = N_EDGES // NW
CHUNK = 80
EPW_PAD = 10000
NCHUNK = EPW_PAD // CHUNK
NSEG = 5
SEGCHUNK = NCHUNK // NSEG
ZBLK = 64
NZFULL = N_NODES // ZBLK
NPAD = 10240

_sc_mesh = plsc.VectorSubcoreMesh(core_axis_name="c", subcore_axis_name="s")


@functools.partial(
    pl.kernel,
    out_type=jax.ShapeDtypeStruct((NC, NPAD), jnp.float32),
    mesh=_sc_mesh,
    scratch_types=[
        pltpu.VMEM((SEGCHUNK, CHUNK), jnp.int32),
        pltpu.VMEM((SEGCHUNK, CHUNK), jnp.float32),
        pltpu.VMEM((NPAD // NS,), jnp.float32),
        pltpu.VMEM_SHARED((NPAD,), jnp.float32),
    ],
)
def _deg_sc(src_hbm, ew_hbm, out_hbm, src_t, ew_t, zbuf, bins):
    cid = lax.axis_index("c")
    sid = lax.axis_index("s")
    wid = sid * NC + cid
    stripe = NPAD // NS

    zeros16 = jnp.zeros((16,), jnp.float32)

    def zrow(i, _):
        zbuf[pl.ds(i * 16, 16)] = zeros16
        return 0

    lax.fori_loop(0, stripe // 16, zrow, 0)
    soff = pl.multiple_of(sid * stripe, stripe)
    pltpu.sync_copy(zbuf, bins.at[pl.ds(soff, stripe)])
    plsc.subcore_barrier()

    for seg in range(NSEG):
        pltpu.sync_copy(src_hbm.at[wid, seg], src_t)
        pltpu.sync_copy(ew_hbm.at[wid, seg], ew_t)

        def chunk_body(c, _):
            pltpu.sync_copy(ew_t.at[c], bins.at[src_t.at[c]], add=True)
            return 0

        lax.fori_loop(0, SEGCHUNK, chunk_body, 0)

    plsc.subcore_barrier()
    pltpu.sync_copy(bins.at[pl.ds(soff, stripe)],
                    out_hbm.at[cid, pl.ds(soff, stripe)])


@functools.partial(
    pl.kernel,
    out_type=jax.ShapeDtypeStruct((NW, NSEG, SEGCHUNK, CHUNK), jnp.float32),
    mesh=_sc_mesh,
    scratch_types=[
        pltpu.VMEM((SEGCHUNK, CHUNK), jnp.int32),
        pltpu.VMEM((SEGCHUNK, CHUNK), jnp.int32),
        pltpu.VMEM((SEGCHUNK, CHUNK), jnp.float32),
        pltpu.VMEM((CHUNK,), jnp.float32),
        pltpu.VMEM((CHUNK,), jnp.float32),
        pltpu.VMEM((SEGCHUNK, CHUNK), jnp.float32),
        pltpu.SemaphoreType.DMA,
        pltpu.SemaphoreType.DMA,
    ],
)
def _w_sc(dinv_hbm, src_hbm, dst_hbm, ew_hbm, w_hbm,
          src_t, dst_t, ew_t, dsrc_t, ddst_t, wout_t, sem1, sem2):
    cid = lax.axis_index("c")
    sid = lax.axis_index("s")
    wid = sid * NC + cid

    for seg in range(NSEG):
        pltpu.sync_copy(src_hbm.at[wid, seg], src_t)
        pltpu.sync_copy(dst_hbm.at[wid, seg], dst_t)
        pltpu.sync_copy(ew_hbm.at[wid, seg], ew_t)

        def chunk_body(c, _):
            d1 = pltpu.async_copy(dinv_hbm.at[src_t.at[c]], dsrc_t, sem1)
            d2 = pltpu.async_copy(dinv_hbm.at[dst_t.at[c]], ddst_t, sem2)
            d1.wait()
            d2.wait()

            def group_body(g, _):
                sl = pl.ds(g * 16, 16)
                wout_t[c, sl] = -(dsrc_t[sl] * ew_t[c, sl] * ddst_t[sl])
                return 0

            lax.fori_loop(0, CHUNK // 16, group_body, 0)
            return 0

        lax.fori_loop(0, SEGCHUNK, chunk_body, 0)
        pltpu.sync_copy(wout_t, w_hbm.at[wid, seg])


@functools.partial(
    pl.kernel,
    out_type=jax.ShapeDtypeStruct((NC, N_NODES, HID), jnp.float32),
    mesh=_sc_mesh,
    scratch_types=[
        pltpu.VMEM((SEGCHUNK, CHUNK), jnp.int32),
        pltpu.VMEM((SEGCHUNK, CHUNK), jnp.int32),
        pltpu.VMEM((SEGCHUNK, CHUNK), jnp.float32),
        pltpu.VMEM((CHUNK, HID), jnp.float32),
        pltpu.VMEM_SHARED((N_NODES, HID), jnp.float32),
        pltpu.SemaphoreType.DMA,
    ],
)
def _lap_sc(v_hbm, src_hbm, dst_hbm, w_hbm, out_hbm,
            src_t, dst_t, w_t, rows, acc, sem):
    cid = lax.axis_index("c")
    sid = lax.axis_index("s")
    wid = sid * NC + cid

    zeros16 = jnp.zeros((16,), jnp.float32)

    def zrow(i, _):
        for k in range(HID // 16):
            rows[i, pl.ds(k * 16, 16)] = zeros16
        return 0

    lax.fori_loop(0, ZBLK, zrow, 0)
    for j in range((NZFULL + NS) // NS):
        b = sid + j * NS

        @pl.when(b < NZFULL)
        def _():
            off = pl.multiple_of(b * ZBLK, ZBLK)
            pltpu.sync_copy(rows.at[pl.ds(0, ZBLK)], acc.at[pl.ds(off, ZBLK)])

        @pl.when(b == NZFULL)
        def _():
            pltpu.sync_copy(rows.at[pl.ds(0, 16)],
                            acc.at[pl.ds(NZFULL * ZBLK, 16)])

    plsc.subcore_barrier()

    for seg in range(NSEG):
        pltpu.sync_copy(src_hbm.at[wid, seg], src_t)
        pltpu.sync_copy(dst_hbm.at[wid, seg], dst_t)
        pltpu.sync_copy(w_hbm.at[wid, seg], w_t)

        def chunk_body(c, _):
            pltpu.async_copy(v_hbm.at[src_t.at[c]], rows, sem).wait()

            def group_body(g, _):
                wv = w_t[c, pl.ds(g * 16, 16)]
                for j in range(16):
                    we = wv[j]
                    e = g * 16 + j
                    for k in range(HID // 16):
                        sl = pl.ds(k * 16, 16)
                        rows[e, sl] = rows[e, sl] * we
                return 0

            lax.fori_loop(0, CHUNK // 16, group_body, 0)
            pltpu.sync_copy(rows, acc.at[dst_t.at[c]], add=True)
            return 0

        lax.fori_loop(0, SEGCHUNK, chunk_body, 0)

    plsc.subcore_barrier()

    for j in range((NZFULL + NS) // NS):
        b = sid + j * NS

        @pl.when(b < NZFULL)
        def _():
            off = pl.multiple_of(b * ZBLK, ZBLK)
            pltpu.sync_copy(acc.at[pl.ds(off, ZBLK)],
                            out_hbm.at[cid, pl.ds(off, ZBLK)])

        @pl.when(b == NZFULL)
        def _():
            pltpu.sync_copy(acc.at[pl.ds(NZFULL * ZBLK, 16)],
                            out_hbm.at[cid, pl.ds(NZFULL * ZBLK, 16)])


def _gru_zr_body(gxzr_ref, hmm_ref, h_ref, z_ref, hr_ref):
    zr = jax.nn.sigmoid(gxzr_ref[...] + hmm_ref[...])
    z = zr[:, :HID]
    r = zr[:, HID:]
    z_ref[...] = z
    hr_ref[...] = h_ref[...] * r


def _gru_h_body(gxh_ref, hrmm_ref, z_ref, h_ref, out_ref):
    htil = jnp.tanh(gxh_ref[...] + hrmm_ref[...])
    z = z_ref[...]
    out_ref[...] = z * h_ref[...] + (1.0 - z) * htil


def _row_spec(c):
    return pl.BlockSpec((ROW_BLK, c), lambda i: (i, 0))


_gru_zr = pl.pallas_call(
    _gru_zr_body,
    grid=(N_NODES // ROW_BLK,),
    in_specs=[_row_spec(2 * HID), _row_spec(2 * HID), _row_spec(HID)],
    out_specs=[_row_spec(HID), _row_spec(HID)],
    out_shape=[
        jax.ShapeDtypeStruct((N_NODES, HID), jnp.float32),
        jax.ShapeDtypeStruct((N_NODES, HID), jnp.float32),
    ],
)

_gru_h = pl.pallas_call(
    _gru_h_body,
    grid=(N_NODES // ROW_BLK,),
    in_specs=[_row_spec(HID), _row_spec(HID), _row_spec(HID), _row_spec(HID)],
    out_specs=_row_spec(HID),
    out_shape=jax.ShapeDtypeStruct((N_NODES, HID), jnp.float32),
)


def _cat_weights(W):
    return jnp.concatenate([W[0] - W[2], W[1], 2.0 * W[2]], axis=0)


def _edge_layout(a, fill):
    a = a.reshape(NW, EPW)
    pad = jnp.full((NW, EPW_PAD - EPW), fill, a.dtype)
    return jnp.concatenate([a, pad], axis=1).reshape(NW, NSEG, SEGCHUNK, CHUNK)


def kernel(X, edge_index, edge_weight, Wxz, bxz, Whz, bhz, Wxr, bxr, Whr, bhr,
           Wxh, bxh, Whh, bhh):
    src = edge_index[0]
    dst = edge_index[1]

    src4 = _edge_layout(src, 0)
    dst4 = _edge_layout(dst, 0)
    ew4 = _edge_layout(edge_weight, 0.0)

    degp = _deg_sc(src4, ew4)
    deg = degp[0] + degp[1]
    dinv = jnp.where(deg > 0, deg ** -0.5, 0.0)
    w4 = _w_sc(dinv, src4, dst4, ew4)

    def S(v):
        parts = _lap_sc(v, src4, dst4, w4)
        return parts[0] + parts[1]

    batches, seq_len, num_nodes, _ = X.shape

    Wx_cat = jnp.concatenate(
        [_cat_weights(Wxz), _cat_weights(Wxr), _cat_weights(Wxh)], axis=1)
    bx_cat = jnp.concatenate([bxz, bxr, bxh])
    Whzr_cat = jnp.concatenate([_cat_weights(Whz), _cat_weights(Whr)], axis=1)
    bh_zr = jnp.concatenate([bhz, bhr])
    Whh_cat = _cat_weights(Whh)

    Gx = []
    for t in range(seq_len):
        Xt = X[0, t]
        X1 = S(Xt)
        X2 = S(X1)
        basis = jnp.concatenate([Xt, X1, X2], axis=1)
        Gx.append(basis @ Wx_cat + bx_cat)

    H = jnp.zeros((num_nodes, HID), dtype=X.dtype)
    states = []
    for t in range(seq_len):
        H1 = S(H)
        H2 = S(H1)
        Bh = jnp.concatenate([H, H1, H2], axis=1)
        hmm = Bh @ Whzr_cat + bh_zr
        Z, HR = _gru_zr(Gx[t][:, :2 * HID], hmm, H)
        R1 = S(HR)
        R2 = S(R1)
        Bhr = jnp.concatenate([HR, R1, R2], axis=1)
        hrmm = Bhr @ Whh_cat + bhh
        H = _gru_h(Gx[t][:, 2 * HID:], hrmm, Z, H)
        states.append(H)

    out = jnp.stack(states, axis=0)[None]
    return out, H[None]


# speedup vs baseline: 3.4669x; 1.4899x over previous
import functools

import jax
import jax.numpy as jnp
from jax import lax
from jax.experimental import pallas as pl
from jax.experimental.pallas import tpu as pltpu
from jax.experimental.pallas import tpu_sc as plsc

N_NODES = 10000
IN_CH = 128
HID = 128
ROW_BLK = 1000

N_EDGES = 320000
NC = 2
NS = 16
NW = NC * NS
EPW = N_EDGES // NW
CHUNK = 80
EPW_PAD = 10000
NCHUNK = EPW_PAD // CHUNK
NSEG = 5
SEGCHUNK = NCHUNK // NSEG
ZBLK = 64
NZFULL = N_NODES // ZBLK
NPAD = 10240

_sc_mesh = plsc.VectorSubcoreMesh(core_axis_name="c", subcore_axis_name="s")


@functools.partial(
    pl.kernel,
    out_type=jax.ShapeDtypeStruct((NC, NPAD), jnp.float32),
    mesh=_sc_mesh,
    scratch_types=[
        pltpu.VMEM((SEGCHUNK, CHUNK), jnp.int32),
        pltpu.VMEM((SEGCHUNK, CHUNK), jnp.float32),
        pltpu.VMEM((NPAD // NS,), jnp.float32),
        pltpu.VMEM_SHARED((NPAD,), jnp.float32),
    ],
)
def _deg_sc(src_hbm, ew_hbm, out_hbm, src_t, ew_t, zbuf, bins):
    cid = lax.axis_index("c")
    sid = lax.axis_index("s")
    wid = sid * NC + cid
    stripe = NPAD // NS

    zeros16 = jnp.zeros((16,), jnp.float32)

    def zrow(i, _):
        zbuf[pl.ds(i * 16, 16)] = zeros16
        return 0

    lax.fori_loop(0, stripe // 16, zrow, 0)
    soff = pl.multiple_of(sid * stripe, stripe)
    pltpu.sync_copy(zbuf, bins.at[pl.ds(soff, stripe)])
    plsc.subcore_barrier()

    for seg in range(NSEG):
        pltpu.sync_copy(src_hbm.at[wid, seg], src_t)
        pltpu.sync_copy(ew_hbm.at[wid, seg], ew_t)

        def chunk_body(c, _):
            pltpu.sync_copy(ew_t.at[c], bins.at[src_t.at[c]], add=True)
            return 0

        lax.fori_loop(0, SEGCHUNK, chunk_body, 0)

    plsc.subcore_barrier()
    pltpu.sync_copy(bins.at[pl.ds(soff, stripe)],
                    out_hbm.at[cid, pl.ds(soff, stripe)])


@functools.partial(
    pl.kernel,
    out_type=jax.ShapeDtypeStruct((NW, NSEG, SEGCHUNK, CHUNK), jnp.float32),
    mesh=_sc_mesh,
    scratch_types=[
        pltpu.VMEM((SEGCHUNK, CHUNK), jnp.int32),
        pltpu.VMEM((SEGCHUNK, CHUNK), jnp.int32),
        pltpu.VMEM((SEGCHUNK, CHUNK), jnp.float32),
        pltpu.VMEM((CHUNK,), jnp.float32),
        pltpu.VMEM((CHUNK,), jnp.float32),
        pltpu.VMEM((SEGCHUNK, CHUNK), jnp.float32),
        pltpu.SemaphoreType.DMA,
        pltpu.SemaphoreType.DMA,
    ],
)
def _w_sc(dinv_hbm, src_hbm, dst_hbm, ew_hbm, w_hbm,
          src_t, dst_t, ew_t, dsrc_t, ddst_t, wout_t, sem1, sem2):
    cid = lax.axis_index("c")
    sid = lax.axis_index("s")
    wid = sid * NC + cid

    for seg in range(NSEG):
        pltpu.sync_copy(src_hbm.at[wid, seg], src_t)
        pltpu.sync_copy(dst_hbm.at[wid, seg], dst_t)
        pltpu.sync_copy(ew_hbm.at[wid, seg], ew_t)

        def chunk_body(c, _):
            d1 = pltpu.async_copy(dinv_hbm.at[src_t.at[c]], dsrc_t, sem1)
            d2 = pltpu.async_copy(dinv_hbm.at[dst_t.at[c]], ddst_t, sem2)
            d1.wait()
            d2.wait()

            def group_body(g, _):
                sl = pl.ds(g * 16, 16)
                wout_t[c, sl] = -(dsrc_t[sl] * ew_t[c, sl] * ddst_t[sl])
                return 0

            lax.fori_loop(0, CHUNK // 16, group_body, 0)
            return 0

        lax.fori_loop(0, SEGCHUNK, chunk_body, 0)
        pltpu.sync_copy(wout_t, w_hbm.at[wid, seg])


@functools.partial(
    pl.kernel,
    out_type=jax.ShapeDtypeStruct((NC, N_NODES, HID), jnp.float32),
    mesh=_sc_mesh,
    scratch_types=[
        pltpu.VMEM((NCHUNK, CHUNK), jnp.int32),
        pltpu.VMEM((SEGCHUNK, CHUNK), jnp.int32),
        pltpu.VMEM((SEGCHUNK, CHUNK), jnp.float32),
        pltpu.VMEM((CHUNK, HID), jnp.float32),
        pltpu.VMEM((CHUNK, HID), jnp.float32),
        pltpu.VMEM_SHARED((N_NODES, HID), jnp.float32),
        pltpu.SemaphoreType.DMA,
        pltpu.SemaphoreType.DMA,
    ],
)
def _lap_sc(v_hbm, srcf_hbm, dst_hbm, w_hbm, out_hbm,
            src_t, dst_t, w_t, rowsA, rowsB, acc, semA, semB):
    cid = lax.axis_index("c")
    sid = lax.axis_index("s")
    wid = sid * NC + cid
    rows = (rowsA, rowsB)
    sems = (semA, semB)

    zeros16 = jnp.zeros((16,), jnp.float32)

    def zrow(i, _):
        for k in range(HID // 16):
            rowsA[i, pl.ds(k * 16, 16)] = zeros16
        return 0

    lax.fori_loop(0, ZBLK, zrow, 0)
    for j in range((NZFULL + NS) // NS):
        b = sid + j * NS

        @pl.when(b < NZFULL)
        def _():
            off = pl.multiple_of(b * ZBLK, ZBLK)
            pltpu.sync_copy(rowsA.at[pl.ds(0, ZBLK)], acc.at[pl.ds(off, ZBLK)])

        @pl.when(b == NZFULL)
        def _():
            pltpu.sync_copy(rowsA.at[pl.ds(0, 16)],
                            acc.at[pl.ds(NZFULL * ZBLK, 16)])

    plsc.subcore_barrier()
    pltpu.sync_copy(srcf_hbm.at[wid], src_t)

    def g_issue(gc, p):
        pltpu.async_copy(v_hbm.at[src_t.at[gc]], rows[p], sems[p])

    def g_wait(p):
        pltpu.make_async_copy(v_hbm.at[src_t.at[0]], rows[p], sems[p]).wait()

    def scale_scatter(c, p):
        def group_body(g, _):
            wv = w_t[c, pl.ds(g * 16, 16)]
            for j in range(16):
                we = wv[j]
                e = g * 16 + j
                for k in range(HID // 16):
                    sl = pl.ds(k * 16, 16)
                    rows[p][e, sl] = rows[p][e, sl] * we
            return 0

        lax.fori_loop(0, CHUNK // 16, group_body, 0)
        pltpu.sync_copy(rows[p], acc.at[dst_t.at[c]], add=True)

    for seg in range(NSEG):
        g0 = seg * SEGCHUNK
        p0 = g0 % 2
        pltpu.sync_copy(dst_hbm.at[wid, seg], dst_t)
        pltpu.sync_copy(w_hbm.at[wid, seg], w_t)
        g_issue(g0, p0)

        def pair_body(pr, _):
            c0 = 2 * pr
            g_wait(p0)
            g_issue(g0 + c0 + 1, 1 - p0)
            scale_scatter(c0, p0)
            g_wait(1 - p0)

            @pl.when(c0 + 2 < SEGCHUNK)
            def _():
                g_issue(g0 + c0 + 2, p0)

            scale_scatter(c0 + 1, 1 - p0)
            return 0

        lax.fori_loop(0, SEGCHUNK // 2, pair_body, 0)
        g_wait(p0)
        scale_scatter(SEGCHUNK - 1, p0)

    plsc.subcore_barrier()

    for j in range((NZFULL + NS) // NS):
        b = sid + j * NS

        @pl.when(b < NZFULL)
        def _():
            off = pl.multiple_of(b * ZBLK, ZBLK)
            pltpu.sync_copy(acc.at[pl.ds(off, ZBLK)],
                            out_hbm.at[cid, pl.ds(off, ZBLK)])

        @pl.when(b == NZFULL)
        def _():
            pltpu.sync_copy(acc.at[pl.ds(NZFULL * ZBLK, 16)],
                            out_hbm.at[cid, pl.ds(NZFULL * ZBLK, 16)])


def _gru_zr_body(gxzr_ref, hmm_ref, h_ref, z_ref, hr_ref):
    zr = jax.nn.sigmoid(gxzr_ref[...] + hmm_ref[...])
    z = zr[:, :HID]
    r = zr[:, HID:]
    z_ref[...] = z
    hr_ref[...] = h_ref[...] * r


def _gru_h_body(gxh_ref, hrmm_ref, z_ref, h_ref, out_ref):
    htil = jnp.tanh(gxh_ref[...] + hrmm_ref[...])
    z = z_ref[...]
    out_ref[...] = z * h_ref[...] + (1.0 - z) * htil


def _row_spec(c):
    return pl.BlockSpec((ROW_BLK, c), lambda i: (i, 0))


_gru_zr = pl.pallas_call(
    _gru_zr_body,
    grid=(N_NODES // ROW_BLK,),
    in_specs=[_row_spec(2 * HID), _row_spec(2 * HID), _row_spec(HID)],
    out_specs=[_row_spec(HID), _row_spec(HID)],
    out_shape=[
        jax.ShapeDtypeStruct((N_NODES, HID), jnp.float32),
        jax.ShapeDtypeStruct((N_NODES, HID), jnp.float32),
    ],
)

_gru_h = pl.pallas_call(
    _gru_h_body,
    grid=(N_NODES // ROW_BLK,),
    in_specs=[_row_spec(HID), _row_spec(HID), _row_spec(HID), _row_spec(HID)],
    out_specs=_row_spec(HID),
    out_shape=jax.ShapeDtypeStruct((N_NODES, HID), jnp.float32),
)


def _cat_weights(W):
    return jnp.concatenate([W[0] - W[2], W[1], 2.0 * W[2]], axis=0)


def _edge_layout(a, fill):
    a = a.reshape(NW, EPW)
    pad = jnp.full((NW, EPW_PAD - EPW), fill, a.dtype)
    return jnp.concatenate([a, pad], axis=1).reshape(NW, NSEG, SEGCHUNK, CHUNK)


def kernel(X, edge_index, edge_weight, Wxz, bxz, Whz, bhz, Wxr, bxr, Whr, bhr,
           Wxh, bxh, Whh, bhh):
    src = edge_index[0]
    dst = edge_index[1]

    src4 = _edge_layout(src, 0)
    dst4 = _edge_layout(dst, 0)
    ew4 = _edge_layout(edge_weight, 0.0)
    srcf = src4.reshape(NW, NCHUNK, CHUNK)

    degp = _deg_sc(src4, ew4)
    deg = degp[0] + degp[1]
    dinv = jnp.where(deg > 0, deg ** -0.5, 0.0)
    w4 = _w_sc(dinv, src4, dst4, ew4)

    def S(v):
        parts = _lap_sc(v, srcf, dst4, w4)
        return parts[0] + parts[1]

    batches, seq_len, num_nodes, _ = X.shape

    Wx_cat = jnp.concatenate(
        [_cat_weights(Wxz), _cat_weights(Wxr), _cat_weights(Wxh)], axis=1)
    bx_cat = jnp.concatenate([bxz, bxr, bxh])
    Whzr_cat = jnp.concatenate([_cat_weights(Whz), _cat_weights(Whr)], axis=1)
    bh_zr = jnp.concatenate([bhz, bhr])
    Whh_cat = _cat_weights(Whh)

    Gx = []
    for t in range(seq_len):
        Xt = X[0, t]
        X1 = S(Xt)
        X2 = S(X1)
        basis = jnp.concatenate([Xt, X1, X2], axis=1)
        Gx.append(basis @ Wx_cat + bx_cat)

    H = jnp.zeros((num_nodes, HID), dtype=X.dtype)
    states = []
    for t in range(seq_len):
        H1 = S(H)
        H2 = S(H1)
        Bh = jnp.concatenate([H, H1, H2], axis=1)
        hmm = Bh @ Whzr_cat + bh_zr
        Z, HR = _gru_zr(Gx[t][:, :2 * HID], hmm, H)
        R1 = S(HR)
        R2 = S(R1)
        Bhr = jnp.concatenate([HR, R1, R2], axis=1)
        hrmm = Bhr @ Whh_cat + bhh
        H = _gru_h(Gx[t][:, 2 * HID:], hrmm, Z, H)
        states.append(H)

    out = jnp.stack(states, axis=0)[None]
    return out, H[None]
